# baseline (speedup 1.0000x reference)
import functools

import jax
import jax.numpy as jnp
from jax import lax
from jax.experimental import pallas as pl
from jax.experimental.pallas import tpu as pltpu
from jax.experimental.pallas import tpu_sc as plsc

D = 64
B = 4096 * 200
NW = 32
B_PER_W = B // NW
CHUNK = 128
N_CHUNK = B_PER_W // CHUNK
N_PAIR = N_CHUNK // 2

_MESH = plsc.VectorSubcoreMesh(core_axis_name="c", subcore_axis_name="s")


@functools.partial(
    pl.kernel,
    out_type=jax.ShapeDtypeStruct((B, D), jnp.float32),
    mesh=_MESH,
    scratch_types=[
        pltpu.VMEM((N_CHUNK, CHUNK), jnp.int32),
        pltpu.VMEM((CHUNK, D), jnp.float32),
        pltpu.VMEM((CHUNK, D), jnp.float32),
        pltpu.SemaphoreType.DMA,
        pltpu.SemaphoreType.DMA,
    ],
    compiler_params=pltpu.CompilerParams(use_tc_tiling_on_sc=False),
)
def _emb(idx_hbm, table_hbm, out_hbm, idx_v, rows0, rows1, sem0, sem1):
    wid = lax.axis_index("s") * 2 + lax.axis_index("c")
    base = wid * B_PER_W
    pltpu.sync_copy(idx_hbm.at[wid], idx_v)

    pltpu.async_copy(table_hbm.at[idx_v.at[0]], rows0, sem0)

    def body(i, _):
        j0 = i * 2
        pltpu.make_async_copy(table_hbm.at[idx_v.at[j0]], rows0, sem0).wait()
        pltpu.async_copy(table_hbm.at[idx_v.at[j0 + 1]], rows1, sem1)
        pltpu.sync_copy(rows0, out_hbm.at[pl.ds(base + j0 * CHUNK, CHUNK)])

        pltpu.make_async_copy(table_hbm.at[idx_v.at[j0 + 1]], rows1, sem1).wait()

        @pl.when(i + 1 < N_PAIR)
        def _():
            pltpu.async_copy(table_hbm.at[idx_v.at[j0 + 2]], rows0, sem0)

        pltpu.sync_copy(rows1, out_hbm.at[pl.ds(base + (j0 + 1) * CHUNK, CHUNK)])
        return 0

    lax.fori_loop(0, N_PAIR, body, 0)


def kernel(x, table):
    idx = x.reshape(NW, N_CHUNK, CHUNK)
    out = _emb(idx, table)
    return out.reshape(x.shape[0], x.shape[1], D)
